# baseline (speedup 1.0000x reference)
import jax
import jax.numpy as jnp
from jax import lax
from jax.experimental import pallas as pl
from jax.experimental.pallas import tpu as pltpu


def _round_up(x, m):
    return (x + m - 1) // m * m


def _make_kernel(t_chunk, n_chunks, b, hidden, unroll):
    B, H = b, hidden

    def _cell(zx_scr, row, whh_ref, c_old, h_bf16):
        def g(k):
            return (zx_scr[pl.ds(row, B), pl.ds(k * H, H)]
                    + jnp.dot(h_bf16, whh_ref[:, pl.ds(k * H, H)],
                              preferred_element_type=jnp.float32))

        i_g = jax.nn.sigmoid(g(0))
        f_g = jax.nn.sigmoid(g(1))
        g_g = jnp.tanh(g(2))
        o_g = jax.nn.sigmoid(g(3))
        c_new = f_g * c_old + i_g * g_g
        h_new = o_g * jnp.tanh(c_new)
        return h_new, c_new

    def _body(x_ref, wih0_ref, whh0_ref, b0_ref, wih1_ref, whh1_ref, b1_ref,
              fcw_ref, fcb_ref, h0_ref, c0_ref, hN_ref, cN_ref, out_ref,
              hc_scr, y_scr, zx0_scr, zx1_scr):
        c_idx = pl.program_id(0)

        @pl.when(c_idx == 0)
        def _init():
            hc_scr[0] = h0_ref[0]
            hc_scr[1] = c0_ref[0]
            hc_scr[2] = h0_ref[1]
            hc_scr[3] = c0_ref[1]

        @pl.when(c_idx >= 1)
        def _hoist1():
            zx1_scr[...] = (jnp.dot(y_scr[...], wih1_ref[...],
                                    preferred_element_type=jnp.float32)
                            + b1_ref[...])

        @pl.when(c_idx < n_chunks)
        def _hoist0():
            zx0_scr[...] = (jnp.dot(x_ref[...], wih0_ref[...],
                                    preferred_element_type=jnp.float32)
                            + b0_ref[...])

        def l0_part(i, h0, c0):
            row = pl.multiple_of(i * B, 8)
            h0_n, c0_n = _cell(zx0_scr, row, whh0_ref, c0,
                               h0.astype(jnp.bfloat16))
            y_scr[pl.ds(row, B), :] = h0_n.astype(jnp.bfloat16)
            return h0_n, c0_n

        def l1_part(i, h1, c1):
            row = pl.multiple_of(i * B, 8)
            return _cell(zx1_scr, row, whh1_ref, c1, h1.astype(jnp.bfloat16))

        @pl.when(c_idx == 0)
        def _first_chunk():
            h0, c0 = lax.fori_loop(
                0, t_chunk, lambda i, hc: l0_part(i, *hc),
                (hc_scr[0], hc_scr[1]), unroll=unroll)
            hc_scr[0] = h0
            hc_scr[1] = c0

        @pl.when(jnp.logical_and(c_idx >= 1, c_idx < n_chunks))
        def _mid_chunks():
            def step(i, carry):
                h0, c0, h1, c1 = carry
                h0_n, c0_n = l0_part(i, h0, c0)
                h1_n, c1_n = l1_part(i, h1, c1)
                return (h0_n, c0_n, h1_n, c1_n)
            h0, c0, h1, c1 = lax.fori_loop(
                0, t_chunk, step,
                (hc_scr[0], hc_scr[1], hc_scr[2], hc_scr[3]), unroll=unroll)
            hc_scr[0] = h0
            hc_scr[1] = c0
            hc_scr[2] = h1
            hc_scr[3] = c1

        @pl.when(c_idx == n_chunks)
        def _last_chunk():
            h1, c1 = lax.fori_loop(
                0, t_chunk, lambda i, hc: l1_part(i, *hc),
                (hc_scr[2], hc_scr[3]), unroll=unroll)
            hN_ref[0] = hc_scr[0]
            hN_ref[1] = h1
            cN_ref[0] = hc_scr[1]
            cN_ref[1] = c1
            out_ref[...] = (jnp.dot(h1, fcw_ref[...],
                                    preferred_element_type=jnp.float32)
                            + fcb_ref[...])

    return _body


def _impl(embedding, wih_t_0, wih_t_1, whh_t_0, whh_t_1, bias_0, bias_1,
          fc_w_t, fc_b, input_sequence, state_h, state_c, *,
          t_chunk, unroll, single_buffered):
    B, T = input_sequence.shape
    E = embedding.shape[1]
    H = state_h.shape[-1]
    V = fc_w_t.shape[1]

    while T % t_chunk:
        t_chunk //= 2
    n_chunks = T // t_chunk

    wih0 = wih_t_0.astype(jnp.bfloat16)
    whh0 = whh_t_0.astype(jnp.bfloat16)
    wih1 = wih_t_1.astype(jnp.bfloat16)
    whh1 = whh_t_1.astype(jnp.bfloat16)
    b0 = bias_0.astype(jnp.float32)
    b1 = bias_1.astype(jnp.float32)

    V_pad = _round_up(V, 128)
    fcw = fc_w_t
    fcb = fc_b
    if V_pad != V:
        fcw = jnp.pad(fcw, ((0, 0), (0, V_pad - V)))
        fcb = jnp.pad(fcb, ((0, 0), (0, V_pad - V)))
    fcb = fcb.astype(jnp.float32)

    tok = input_sequence.T.reshape(-1)
    x2d = jnp.take(embedding.astype(jnp.bfloat16), tok, axis=0)

    body = _make_kernel(t_chunk, n_chunks, B, H, unroll)

    def const_spec(shape):
        if single_buffered:
            return pl.BlockSpec(shape, lambda c: (0,) * len(shape),
                                pipeline_mode=pl.Buffered(1))
        return pl.BlockSpec(shape, lambda c: (0,) * len(shape))

    in_specs = [
        pl.BlockSpec((t_chunk * B, E),
                     lambda c: (jnp.minimum(c, n_chunks - 1), 0)),
        const_spec(wih0.shape),
        const_spec(whh0.shape),
        const_spec(b0.shape),
        const_spec(wih1.shape),
        const_spec(whh1.shape),
        const_spec(b1.shape),
        const_spec(fcw.shape),
        const_spec(fcb.shape),
        pl.BlockSpec((2, B, H), lambda c: (0, 0, 0)),
        pl.BlockSpec((2, B, H), lambda c: (0, 0, 0)),
    ]
    out_shape = (jax.ShapeDtypeStruct((2, B, H), jnp.float32),
                 jax.ShapeDtypeStruct((2, B, H), jnp.float32),
                 jax.ShapeDtypeStruct((B, V_pad), jnp.float32))
    out_specs = (pl.BlockSpec((2, B, H), lambda c: (0, 0, 0)),
                 pl.BlockSpec((2, B, H), lambda c: (0, 0, 0)),
                 pl.BlockSpec((B, V_pad), lambda c: (0, 0)))
    scratch_shapes = [
        pltpu.VMEM((4, B, H), jnp.float32),
        pltpu.VMEM((t_chunk * B, H), jnp.bfloat16),
        pltpu.VMEM((t_chunk * B, 4 * H), jnp.float32),
        pltpu.VMEM((t_chunk * B, 4 * H), jnp.float32),
    ]

    h_n, c_n, logits = pl.pallas_call(
        body,
        out_shape=out_shape,
        grid_spec=pltpu.PrefetchScalarGridSpec(
            num_scalar_prefetch=0,
            grid=(n_chunks + 1,),
            in_specs=in_specs,
            out_specs=out_specs,
            scratch_shapes=scratch_shapes,
        ),
        compiler_params=pltpu.CompilerParams(
            dimension_semantics=("arbitrary",),
            vmem_limit_bytes=64 * 1024 * 1024,
        ),
    )(x2d, wih0, whh0, b0, wih1, whh1, b1, fcw, fcb,
      state_h.astype(jnp.float32), state_c.astype(jnp.float32))

    return logits[:, :V], (h_n, c_n)


def kernel(embedding, wih_t_0, wih_t_1, whh_t_0, whh_t_1, bias_0, bias_1,
           fc_w_t, fc_b, input_sequence, state_h, state_c):
    return _impl(embedding, wih_t_0, wih_t_1, whh_t_0, whh_t_1, bias_0,
                 bias_1, fc_w_t, fc_b, input_sequence, state_h, state_c,
                 t_chunk=4, unroll=4, single_buffered=True)


# speedup vs baseline: 1.0404x; 1.0404x over previous
import jax
import jax.numpy as jnp
from jax import lax
from jax.experimental import pallas as pl
from jax.experimental.pallas import tpu as pltpu


def _round_up(x, m):
    return (x + m - 1) // m * m


def _make_kernel(t_chunk, n_chunks, b, hidden, unroll):
    B, H = b, hidden

    def _cell(zx_scr, row, whh_ref, c_old, h_bf16):
        def g(k):
            return (zx_scr[pl.ds(row, B), pl.ds(k * H, H)]
                    + jnp.dot(h_bf16, whh_ref[:, pl.ds(k * H, H)],
                              preferred_element_type=jnp.float32))

        i_g = jax.nn.sigmoid(g(0))
        f_g = jax.nn.sigmoid(g(1))
        g_g = jnp.tanh(g(2))
        o_g = jax.nn.sigmoid(g(3))
        c_new = f_g * c_old + i_g * g_g
        h_new = o_g * jnp.tanh(c_new)
        return h_new, c_new

    def _body(x_ref, wih0_ref, whh0_ref, b0_ref, wih1_ref, whh1_ref, b1_ref,
              fcw_ref, fcb_ref, h0_ref, c0_ref, hN_ref, cN_ref, out_ref,
              hc_scr, y_scr, zx0_scr, zx1_scr):
        c_idx = pl.program_id(0)

        @pl.when(c_idx == 0)
        def _init():
            hc_scr[0] = h0_ref[0]
            hc_scr[1] = c0_ref[0]
            hc_scr[2] = h0_ref[1]
            hc_scr[3] = c0_ref[1]

        @pl.when(c_idx >= 1)
        def _hoist1():
            zx1_scr[...] = (jnp.dot(y_scr[...], wih1_ref[...],
                                    preferred_element_type=jnp.float32)
                            + b1_ref[...])

        @pl.when(c_idx < n_chunks)
        def _hoist0():
            zx0_scr[...] = (jnp.dot(x_ref[...], wih0_ref[...],
                                    preferred_element_type=jnp.float32)
                            + b0_ref[...])

        def l0_part(i, h0, c0):
            row = pl.multiple_of(i * B, 8)
            h0_n, c0_n = _cell(zx0_scr, row, whh0_ref, c0,
                               h0.astype(jnp.bfloat16))
            y_scr[pl.ds(row, B), :] = h0_n.astype(jnp.bfloat16)
            return h0_n, c0_n

        def l1_part(i, h1, c1):
            row = pl.multiple_of(i * B, 8)
            return _cell(zx1_scr, row, whh1_ref, c1, h1.astype(jnp.bfloat16))

        @pl.when(c_idx == 0)
        def _first_chunk():
            h0, c0 = lax.fori_loop(
                0, t_chunk, lambda i, hc: l0_part(i, *hc),
                (hc_scr[0], hc_scr[1]), unroll=unroll)
            hc_scr[0] = h0
            hc_scr[1] = c0

        @pl.when(jnp.logical_and(c_idx >= 1, c_idx < n_chunks))
        def _mid_chunks():
            def step(i, carry):
                h0, c0, h1, c1 = carry
                h0_n, c0_n = l0_part(i, h0, c0)
                h1_n, c1_n = l1_part(i, h1, c1)
                return (h0_n, c0_n, h1_n, c1_n)
            h0, c0, h1, c1 = lax.fori_loop(
                0, t_chunk, step,
                (hc_scr[0], hc_scr[1], hc_scr[2], hc_scr[3]), unroll=unroll)
            hc_scr[0] = h0
            hc_scr[1] = c0
            hc_scr[2] = h1
            hc_scr[3] = c1

        @pl.when(c_idx == n_chunks)
        def _last_chunk():
            h1, c1 = lax.fori_loop(
                0, t_chunk, lambda i, hc: l1_part(i, *hc),
                (hc_scr[2], hc_scr[3]), unroll=unroll)
            hN_ref[0] = hc_scr[0]
            hN_ref[1] = h1
            cN_ref[0] = hc_scr[1]
            cN_ref[1] = c1
            out_ref[...] = (jnp.dot(h1, fcw_ref[...],
                                    preferred_element_type=jnp.float32)
                            + fcb_ref[...])

    return _body


def _impl(embedding, wih_t_0, wih_t_1, whh_t_0, whh_t_1, bias_0, bias_1,
          fc_w_t, fc_b, input_sequence, state_h, state_c, *,
          t_chunk, unroll, single_buffered):
    B, T = input_sequence.shape
    E = embedding.shape[1]
    H = state_h.shape[-1]
    V = fc_w_t.shape[1]

    while T % t_chunk:
        t_chunk //= 2
    n_chunks = T // t_chunk

    wih0 = wih_t_0.astype(jnp.bfloat16)
    whh0 = whh_t_0.astype(jnp.bfloat16)
    wih1 = wih_t_1.astype(jnp.bfloat16)
    whh1 = whh_t_1.astype(jnp.bfloat16)
    b0 = bias_0.astype(jnp.float32)
    b1 = bias_1.astype(jnp.float32)

    V_pad = _round_up(V, 128)
    fcw = fc_w_t
    fcb = fc_b
    if V_pad != V:
        fcw = jnp.pad(fcw, ((0, 0), (0, V_pad - V)))
        fcb = jnp.pad(fcb, ((0, 0), (0, V_pad - V)))
    fcb = fcb.astype(jnp.float32)

    tok = input_sequence.T.reshape(-1)
    x2d = jnp.take(embedding.astype(jnp.bfloat16), tok, axis=0)

    body = _make_kernel(t_chunk, n_chunks, B, H, unroll)

    def const_spec(shape):
        if single_buffered:
            return pl.BlockSpec(shape, lambda c: (0,) * len(shape),
                                pipeline_mode=pl.Buffered(1))
        return pl.BlockSpec(shape, lambda c: (0,) * len(shape))

    in_specs = [
        pl.BlockSpec((t_chunk * B, E),
                     lambda c: (jnp.minimum(c, n_chunks - 1), 0)),
        const_spec(wih0.shape),
        const_spec(whh0.shape),
        const_spec(b0.shape),
        const_spec(wih1.shape),
        const_spec(whh1.shape),
        const_spec(b1.shape),
        const_spec(fcw.shape),
        const_spec(fcb.shape),
        pl.BlockSpec((2, B, H), lambda c: (0, 0, 0)),
        pl.BlockSpec((2, B, H), lambda c: (0, 0, 0)),
    ]
    out_shape = (jax.ShapeDtypeStruct((2, B, H), jnp.float32),
                 jax.ShapeDtypeStruct((2, B, H), jnp.float32),
                 jax.ShapeDtypeStruct((B, V_pad), jnp.float32))
    out_specs = (pl.BlockSpec((2, B, H), lambda c: (0, 0, 0)),
                 pl.BlockSpec((2, B, H), lambda c: (0, 0, 0)),
                 pl.BlockSpec((B, V_pad), lambda c: (0, 0)))
    scratch_shapes = [
        pltpu.VMEM((4, B, H), jnp.float32),
        pltpu.VMEM((t_chunk * B, H), jnp.bfloat16),
        pltpu.VMEM((t_chunk * B, 4 * H), jnp.float32),
        pltpu.VMEM((t_chunk * B, 4 * H), jnp.float32),
    ]

    h_n, c_n, logits = pl.pallas_call(
        body,
        out_shape=out_shape,
        grid_spec=pltpu.PrefetchScalarGridSpec(
            num_scalar_prefetch=0,
            grid=(n_chunks + 1,),
            in_specs=in_specs,
            out_specs=out_specs,
            scratch_shapes=scratch_shapes,
        ),
        compiler_params=pltpu.CompilerParams(
            dimension_semantics=("arbitrary",),
            vmem_limit_bytes=64 * 1024 * 1024,
        ),
    )(x2d, wih0, whh0, b0, wih1, whh1, b1, fcw, fcb,
      state_h.astype(jnp.float32), state_c.astype(jnp.float32))

    return logits[:, :V], (h_n, c_n)


def kernel(embedding, wih_t_0, wih_t_1, whh_t_0, whh_t_1, bias_0, bias_1,
           fc_w_t, fc_b, input_sequence, state_h, state_c):
    return _impl(embedding, wih_t_0, wih_t_1, whh_t_0, whh_t_1, bias_0,
                 bias_1, fc_w_t, fc_b, input_sequence, state_h, state_c,
                 t_chunk=8, unroll=8, single_buffered=True)


# speedup vs baseline: 1.0883x; 1.0460x over previous
import jax
import jax.numpy as jnp
from jax import lax
from jax.experimental import pallas as pl
from jax.experimental.pallas import tpu as pltpu


def _round_up(x, m):
    return (x + m - 1) // m * m


def _make_kernel(t_chunk, n_chunks, b, hidden, unroll):
    B, H = b, hidden

    def _cell(zx_scr, row, whh_ref, c_old, h_bf16):
        def g(k):
            return (zx_scr[pl.ds(row, B), pl.ds(k * H, H)]
                    + jnp.dot(h_bf16, whh_ref[:, pl.ds(k * H, H)],
                              preferred_element_type=jnp.float32))

        i_g = jax.nn.sigmoid(g(0))
        f_g = jax.nn.sigmoid(g(1))
        g_g = jnp.tanh(g(2))
        o_g = jax.nn.sigmoid(g(3))
        c_new = f_g * c_old + i_g * g_g
        h_new = o_g * jnp.tanh(c_new)
        return h_new, c_new

    def _body(x_ref, wih0_ref, whh0_ref, b0_ref, wih1_ref, whh1_ref, b1_ref,
              fcw_ref, fcb_ref, h0_ref, c0_ref, hN_ref, cN_ref, out_ref,
              hc_scr, y_scr, zx0_scr, zx1_scr):
        c_idx = pl.program_id(0)

        @pl.when(c_idx == 0)
        def _init():
            hc_scr[0] = h0_ref[0]
            hc_scr[1] = c0_ref[0]
            hc_scr[2] = h0_ref[1]
            hc_scr[3] = c0_ref[1]

        @pl.when(c_idx >= 1)
        def _hoist1():
            zx1_scr[...] = (jnp.dot(y_scr[...], wih1_ref[...],
                                    preferred_element_type=jnp.float32)
                            + b1_ref[...])

        @pl.when(c_idx < n_chunks)
        def _hoist0():
            zx0_scr[...] = (jnp.dot(x_ref[...], wih0_ref[...],
                                    preferred_element_type=jnp.float32)
                            + b0_ref[...])

        def l0_part(i, h0, c0):
            row = pl.multiple_of(i * B, 8)
            h0_n, c0_n = _cell(zx0_scr, row, whh0_ref, c0,
                               h0.astype(jnp.bfloat16))
            y_scr[pl.ds(row, B), :] = h0_n.astype(jnp.bfloat16)
            return h0_n, c0_n

        def l1_part(i, h1, c1):
            row = pl.multiple_of(i * B, 8)
            return _cell(zx1_scr, row, whh1_ref, c1, h1.astype(jnp.bfloat16))

        @pl.when(c_idx == 0)
        def _first_chunk():
            h0, c0 = lax.fori_loop(
                0, t_chunk, lambda i, hc: l0_part(i, *hc),
                (hc_scr[0], hc_scr[1]), unroll=unroll)
            hc_scr[0] = h0
            hc_scr[1] = c0

        @pl.when(jnp.logical_and(c_idx >= 1, c_idx < n_chunks))
        def _mid_chunks():
            def step(i, carry):
                h0, c0, h1, c1 = carry
                h0_n, c0_n = l0_part(i, h0, c0)
                h1_n, c1_n = l1_part(i, h1, c1)
                return (h0_n, c0_n, h1_n, c1_n)
            h0, c0, h1, c1 = lax.fori_loop(
                0, t_chunk, step,
                (hc_scr[0], hc_scr[1], hc_scr[2], hc_scr[3]), unroll=unroll)
            hc_scr[0] = h0
            hc_scr[1] = c0
            hc_scr[2] = h1
            hc_scr[3] = c1

        @pl.when(c_idx == n_chunks)
        def _last_chunk():
            h1, c1 = lax.fori_loop(
                0, t_chunk, lambda i, hc: l1_part(i, *hc),
                (hc_scr[2], hc_scr[3]), unroll=unroll)
            hN_ref[0] = hc_scr[0]
            hN_ref[1] = h1
            cN_ref[0] = hc_scr[1]
            cN_ref[1] = c1
            out_ref[...] = (jnp.dot(h1, fcw_ref[...],
                                    preferred_element_type=jnp.float32)
                            + fcb_ref[...])

    return _body


def _impl(embedding, wih_t_0, wih_t_1, whh_t_0, whh_t_1, bias_0, bias_1,
          fc_w_t, fc_b, input_sequence, state_h, state_c, *,
          t_chunk, unroll, single_buffered):
    B, T = input_sequence.shape
    E = embedding.shape[1]
    H = state_h.shape[-1]
    V = fc_w_t.shape[1]

    while T % t_chunk:
        t_chunk //= 2
    n_chunks = T // t_chunk

    wih0 = wih_t_0.astype(jnp.bfloat16)
    whh0 = whh_t_0.astype(jnp.bfloat16)
    wih1 = wih_t_1.astype(jnp.bfloat16)
    whh1 = whh_t_1.astype(jnp.bfloat16)
    b0 = bias_0.astype(jnp.float32)
    b1 = bias_1.astype(jnp.float32)

    V_pad = _round_up(V, 128)
    fcw = fc_w_t
    fcb = fc_b
    if V_pad != V:
        fcw = jnp.pad(fcw, ((0, 0), (0, V_pad - V)))
        fcb = jnp.pad(fcb, ((0, 0), (0, V_pad - V)))
    fcb = fcb.astype(jnp.float32)

    tok = input_sequence.T.reshape(-1)
    x2d = embedding.astype(jnp.bfloat16).at[tok].get(
        mode="promise_in_bounds")

    body = _make_kernel(t_chunk, n_chunks, B, H, unroll)

    def const_spec(shape):
        if single_buffered:
            return pl.BlockSpec(shape, lambda c: (0,) * len(shape),
                                pipeline_mode=pl.Buffered(1))
        return pl.BlockSpec(shape, lambda c: (0,) * len(shape))

    in_specs = [
        pl.BlockSpec((t_chunk * B, E),
                     lambda c: (jnp.minimum(c, n_chunks - 1), 0)),
        const_spec(wih0.shape),
        const_spec(whh0.shape),
        const_spec(b0.shape),
        const_spec(wih1.shape),
        const_spec(whh1.shape),
        const_spec(b1.shape),
        const_spec(fcw.shape),
        const_spec(fcb.shape),
        pl.BlockSpec((2, B, H), lambda c: (0, 0, 0)),
        pl.BlockSpec((2, B, H), lambda c: (0, 0, 0)),
    ]
    out_shape = (jax.ShapeDtypeStruct((2, B, H), jnp.float32),
                 jax.ShapeDtypeStruct((2, B, H), jnp.float32),
                 jax.ShapeDtypeStruct((B, V_pad), jnp.float32))
    out_specs = (pl.BlockSpec((2, B, H), lambda c: (0, 0, 0)),
                 pl.BlockSpec((2, B, H), lambda c: (0, 0, 0)),
                 pl.BlockSpec((B, V_pad), lambda c: (0, 0)))
    scratch_shapes = [
        pltpu.VMEM((4, B, H), jnp.float32),
        pltpu.VMEM((t_chunk * B, H), jnp.bfloat16),
        pltpu.VMEM((t_chunk * B, 4 * H), jnp.float32),
        pltpu.VMEM((t_chunk * B, 4 * H), jnp.float32),
    ]

    h_n, c_n, logits = pl.pallas_call(
        body,
        out_shape=out_shape,
        grid_spec=pltpu.PrefetchScalarGridSpec(
            num_scalar_prefetch=0,
            grid=(n_chunks + 1,),
            in_specs=in_specs,
            out_specs=out_specs,
            scratch_shapes=scratch_shapes,
        ),
        compiler_params=pltpu.CompilerParams(
            dimension_semantics=("arbitrary",),
            vmem_limit_bytes=64 * 1024 * 1024,
        ),
    )(x2d, wih0, whh0, b0, wih1, whh1, b1, fcw, fcb,
      state_h.astype(jnp.float32), state_c.astype(jnp.float32))

    return logits[:, :V], (h_n, c_n)


def kernel(embedding, wih_t_0, wih_t_1, whh_t_0, whh_t_1, bias_0, bias_1,
           fc_w_t, fc_b, input_sequence, state_h, state_c):
    return _impl(embedding, wih_t_0, wih_t_1, whh_t_0, whh_t_1, bias_0,
                 bias_1, fc_w_t, fc_b, input_sequence, state_h, state_c,
                 t_chunk=8, unroll=8, single_buffered=True)
